# baseline (speedup 1.0000x reference)
import functools

import jax
import jax.numpy as jnp
from jax import lax
from jax.experimental import pallas as pl
from jax.experimental.pallas import tpu as pltpu
from jax.experimental.pallas import tpu_sc as plsc

K = 48
ROWS = 16384
N = 2048
NW = 32
RW = ROWS // NW
SLAB = 16
L = 16


def _vsort(k, v):
    return plsc.sort_key_val(k, v)


def _ce(ka, va, kb, vb):
    m = kb < ka
    lo_k = jnp.where(m, kb, ka)
    hi_k = jnp.where(m, ka, kb)
    lo_v = jnp.where(m, vb, va)
    hi_v = jnp.where(m, va, vb)
    return lo_k, lo_v, hi_k, hi_v


def _rev(x):
    return lax.rev(x, (0,))


def _merge_halves(ks, vs):
    n = len(ks) // 2
    rk = [_rev(ks[2 * n - 1 - i]) for i in range(n)]
    rv = [_rev(vs[2 * n - 1 - i]) for i in range(n)]
    lo_k, lo_v, hi_k, hi_v = [], [], [], []
    for i in range(n):
        a, b, c, d = _ce(ks[i], vs[i], rk[i], rv[i])
        lo_k.append(a)
        lo_v.append(b)
        hi_k.append(c)
        hi_v.append(d)
    lo_k, lo_v = _clean(lo_k, lo_v)
    hi_k, hi_v = _clean(hi_k, hi_v)
    return lo_k + hi_k, lo_v + hi_v


def _merge_low(ks, vs):
    n = len(ks) // 2
    rk = [_rev(ks[2 * n - 1 - i]) for i in range(n)]
    rv = [_rev(vs[2 * n - 1 - i]) for i in range(n)]
    lo_k, lo_v = [], []
    for i in range(n):
        a, b, _, _ = _ce(ks[i], vs[i], rk[i], rv[i])
        lo_k.append(a)
        lo_v.append(b)
    return _clean(lo_k, lo_v)


def _clean(ks, vs):
    n = len(ks)
    if n == 1:
        k, v = _vsort(ks[0], vs[0])
        return [k], [v]
    h = n // 2
    lo_k, lo_v, hi_k, hi_v = list(ks[:h]), list(vs[:h]), list(ks[h:]), list(vs[h:])
    for i in range(h):
        a, b, c, d = _ce(lo_k[i], lo_v[i], hi_k[i], hi_v[i])
        lo_k[i], lo_v[i], hi_k[i], hi_v[i] = a, b, c, d
    lo_k, lo_v = _clean(lo_k, lo_v)
    hi_k, hi_v = _clean(hi_k, hi_v)
    return lo_k + hi_k, lo_v + hi_v


def _row_topk(slab_v, r, kbuf, vbuf):
    iota = lax.broadcasted_iota(jnp.int32, (L,), 0)

    def l01(j, _):
        k0 = slab_v[r, pl.ds(j * 32, L)]
        k1 = slab_v[r, pl.ds(j * 32 + L, L)]
        v0 = iota + j * 32
        v1 = iota + (j * 32 + L)
        k0, v0 = _vsort(k0, v0)
        k1, v1 = _vsort(k1, v1)
        ks, vs = _merge_halves([k0, k1], [v0, v1])
        for t in range(2):
            kbuf[pl.ds(j * 32 + t * L, L)] = ks[t]
            vbuf[pl.ds(j * 32 + t * L, L)] = vs[t]
        return 0

    lax.fori_loop(0, N // 32, l01, 0)

    def l2(j, _):
        ks = [kbuf[pl.ds(j * 64 + t * L, L)] for t in range(4)]
        vs = [vbuf[pl.ds(j * 64 + t * L, L)] for t in range(4)]
        ks, vs = _merge_halves(ks[:2] + ks[2:], vs[:2] + vs[2:])
        for t in range(4):
            kbuf[pl.ds(j * 64 + t * L, L)] = ks[t]
            vbuf[pl.ds(j * 64 + t * L, L)] = vs[t]
        return 0

    lax.fori_loop(0, N // 64, l2, 0)

    for lvl in range(5):
        stride = 64 << (lvl + 1)

        def lm(j, _, stride=stride):
            base_a = j * stride
            base_b = base_a + (stride // 2)
            ks = [kbuf[pl.ds(base_a + t * L, L)] for t in range(4)]
            vs = [vbuf[pl.ds(base_a + t * L, L)] for t in range(4)]
            ks += [kbuf[pl.ds(base_b + t * L, L)] for t in range(4)]
            vs += [vbuf[pl.ds(base_b + t * L, L)] for t in range(4)]
            ks, vs = _merge_low(ks, vs)
            for t in range(4):
                kbuf[pl.ds(base_a + t * L, L)] = ks[t]
                vbuf[pl.ds(base_a + t * L, L)] = vs[t]
            return 0

        lax.fori_loop(0, N // stride, lm, 0)


def _sc_kernel_body(d_hbm, idx_hbm, val_hbm, slab_v, kbuf, vbuf,
                    oi_v, ov_v, sem):
    wid = lax.axis_index("s") * 2 + lax.axis_index("c")
    base_row = wid * RW

    def slab_loop(s, _):
        row0 = base_row + s * SLAB
        pltpu.sync_copy(d_hbm.at[pl.ds(row0, SLAB)], slab_v)

        def row_loop(r, _):
            _row_topk(slab_v, r, kbuf, vbuf)
            for t in range(K // L):
                ov_v[r, pl.ds(t * L, L)] = kbuf[pl.ds(t * L, L)]
                oi_v[r, pl.ds(t * L, L)] = vbuf[pl.ds(t * L, L)]
            return 0

        lax.fori_loop(0, SLAB, row_loop, 0)
        pltpu.sync_copy(oi_v, idx_hbm.at[pl.ds(row0, SLAB)])
        pltpu.sync_copy(ov_v, val_hbm.at[pl.ds(row0, SLAB)])
        return 0

    lax.fori_loop(0, RW // SLAB, slab_loop, 0)


def kernel(D):
    b, q, n = D.shape
    Df = D.reshape(b * q, n)
    mesh = plsc.VectorSubcoreMesh(core_axis_name="c", subcore_axis_name="s")
    run = pl.kernel(
        _sc_kernel_body,
        mesh=mesh,
        compiler_params=pltpu.CompilerParams(needs_layout_passes=False),
        out_type=[
            jax.ShapeDtypeStruct((ROWS, K), jnp.int32),
            jax.ShapeDtypeStruct((ROWS, K), jnp.float32),
        ],
        scratch_types=[
            pltpu.VMEM((SLAB, N), jnp.float32),
            pltpu.VMEM((N,), jnp.float32),
            pltpu.VMEM((N,), jnp.int32),
            pltpu.VMEM((SLAB, K), jnp.int32),
            pltpu.VMEM((SLAB, K), jnp.float32),
            pltpu.SemaphoreType.DMA,
        ],
    )
    idx, vals = run(Df)
    return idx.reshape(b, q, K), vals.reshape(b, q, K)
